# baseline (speedup 1.0000x reference)
import functools

import jax
import jax.numpy as jnp
from jax import lax
from jax.experimental import pallas as pl
from jax.experimental.pallas import tpu as pltpu
from jax.experimental.pallas import tpu_sc as plsc

B, S, D = 4, 8192, 768
ROWS = B * S
NW = 32
RPW = ROWS // NW
CH = 16
NCH = RPW // CH
NL = 16
DV = D // NL


NSLOT = 3


def _pe_body(x_hbm, tb_hbm, pos_hbm, time_hbm, out_hbm,
             xb, tbuf, pb, idxall, semL0, semL1, semL2, semS0, semS1, semS2):
    wid = lax.axis_index("s") * 2 + lax.axis_index("c")
    base = wid * RPW
    sbase = base % S
    semL = (semL0, semL1, semL2)
    semS = (semS0, semS1, semS2)

    pltpu.sync_copy(tb_hbm.at[pl.ds(base, RPW)], idxall)

    def load_descs(i, b):
        r0 = base + i * CH
        p0 = sbase + i * CH
        return (
            pltpu.make_async_copy(x_hbm.at[pl.ds(r0, CH)], xb.at[b], semL[b]),
            pltpu.make_async_copy(pos_hbm.at[pl.ds(p0, CH)], pb.at[b], semL[b]),
            pltpu.make_async_copy(
                time_hbm.at[idxall.at[pl.ds(i * CH, CH)]], tbuf.at[b], semL[b]),
        )

    def store_desc(i, b):
        r0 = base + i * CH
        return pltpu.make_async_copy(xb.at[b], out_hbm.at[pl.ds(r0, CH)],
                                     semS[b])

    def issue_loads(i, b):
        for d in load_descs(i, b):
            d.start()

    issue_loads(0, 0)
    issue_loads(1, 1)

    def chunk(i, b):
        @pl.when(i >= 2)
        def _():
            store_desc(i - 2, (b + 1) % NSLOT).wait()

        @pl.when((i >= 1) & (i + 1 < NCH))
        def _():
            issue_loads(i + 1, (b + 1) % NSLOT)

        for d in load_descs(i, b):
            d.wait()

        x_, t_, p_ = xb.at[b], tbuf.at[b], pb.at[b]

        def row(c, carry):
            for j in range(DV):
                sl = pl.ds(j * NL, NL)
                plsc.addupdate(x_.at[c, sl], t_[c, sl])
                plsc.addupdate(x_.at[c, sl], p_[c, sl])
            return carry

        lax.fori_loop(0, CH, row, None)
        store_desc(i, b).start()

    def outer(g, carry):
        i0 = NSLOT * g
        for b in range(NSLOT):
            chunk(i0 + b, b)
        return carry

    n_full = NCH // NSLOT
    lax.fori_loop(0, n_full, outer, None)
    for i in range(n_full * NSLOT, NCH):
        chunk(i, i % NSLOT)
    for i in (NCH - 2, NCH - 1):
        store_desc(i, i % NSLOT).wait()


@jax.jit
def _pe(x2d, tb1d, pos_table, time_table):
    mesh = plsc.VectorSubcoreMesh(core_axis_name="c", subcore_axis_name="s")
    return pl.kernel(
        _pe_body,
        mesh=mesh,
        out_type=jax.ShapeDtypeStruct((ROWS, D), jnp.float32),
        scratch_types=[
            pltpu.VMEM((NSLOT, CH, D), jnp.float32),
            pltpu.VMEM((NSLOT, CH, D), jnp.float32),
            pltpu.VMEM((NSLOT, CH, D), jnp.float32),
            pltpu.VMEM((RPW,), jnp.int32),
            pltpu.SemaphoreType.DMA,
            pltpu.SemaphoreType.DMA,
            pltpu.SemaphoreType.DMA,
            pltpu.SemaphoreType.DMA,
            pltpu.SemaphoreType.DMA,
            pltpu.SemaphoreType.DMA,
        ],
    )(x2d, tb1d, pos_table, time_table)


def kernel(x, time_buckets, pos_table, time_table):
    x2d = x.reshape(ROWS, D)
    tb1d = time_buckets.astype(jnp.int32).reshape(ROWS)
    out = _pe(x2d, tb1d, pos_table, time_table)
    return out.reshape(B, S, D)
